# Initial kernel scaffold; baseline (speedup 1.0000x reference)
import functools

import jax
import jax.numpy as jnp
from jax import lax
from jax.experimental import pallas as pl
from jax.experimental.pallas import tpu as pltpu
from jax.experimental.pallas import tpu_sc as plsc

NC = 2
NS = 16
NW = NC * NS
K = 128


def _sc_aggregate(x2d, srcp, dstp, ecp, zx, zec, n_pad, e_pad):
    din = x2d.shape[1]
    zr = n_pad // NS
    b_w = e_pad // NW
    n_iter = b_w // K

    mesh = plsc.VectorSubcoreMesh(core_axis_name="c", subcore_axis_name="s")

    @functools.partial(
        pl.kernel,
        out_type=(
            jax.ShapeDtypeStruct((NC, n_pad, din), jnp.float32),
            jax.ShapeDtypeStruct((NC, n_pad, 16), jnp.float32),
        ),
        mesh=mesh,
        scratch_types=[
            pltpu.VMEM((K,), jnp.int32),
            pltpu.VMEM((K,), jnp.int32),
            pltpu.VMEM((K, din), jnp.float32),
            pltpu.VMEM((K, 16), jnp.float32),
            pltpu.VMEM_SHARED((n_pad, din), jnp.float32),
            pltpu.VMEM_SHARED((n_pad, 16), jnp.float32),
            pltpu.SemaphoreType.DMA,
        ],
    )
    def sc_agg(x_hbm, src_hbm, dst_hbm, ec_hbm, zx_hbm, zec_hbm,
               outx_hbm, outec_hbm,
               srcv, dstv, rowsv, ecv, aggx_sh, aggec_sh, sem):
        cid = lax.axis_index("c")
        sid = lax.axis_index("s")
        wid = cid * NS + sid
        r0 = sid * zr
        pltpu.sync_copy(zx_hbm, aggx_sh.at[pl.ds(r0, zr)])
        pltpu.sync_copy(zec_hbm, aggec_sh.at[pl.ds(r0, zr)])
        plsc.subcore_barrier()

        base0 = wid * b_w

        @pl.loop(0, n_iter)
        def _(i):
            base = base0 + i * K
            pltpu.sync_copy(src_hbm.at[pl.ds(base, K)], srcv)
            pltpu.sync_copy(dst_hbm.at[pl.ds(base, K)], dstv)
            pltpu.async_copy(x_hbm.at[srcv], rowsv, sem).wait()
            pltpu.sync_copy(ec_hbm.at[pl.ds(base, K)], ecv)
            pltpu.sync_copy(rowsv, aggx_sh.at[dstv], add=True)
            pltpu.sync_copy(ecv, aggec_sh.at[dstv], add=True)

        plsc.subcore_barrier()
        pltpu.sync_copy(aggx_sh.at[pl.ds(r0, zr)],
                        outx_hbm.at[cid, pl.ds(r0, zr)])
        pltpu.sync_copy(aggec_sh.at[pl.ds(r0, zr)],
                        outec_hbm.at[cid, pl.ds(r0, zr)])

    return sc_agg(x2d, srcp, dstp, ecp, zx, zec)


def _tc_apply(aggx, aggec, x2dp, wmx, wext, wax, wah, ba2, de, bn):
    n_pad, din = x2dp.shape
    dout = wax.shape[1]

    def body(ax_ref, ae_ref, x_ref, wmx_ref, we_ref, wax_ref, wah_ref,
             ba_ref, o_ref):
        sx = ax_ref[0] + ax_ref[1]
        sec = ae_ref[0] + ae_ref[1]
        cnt = sec[:, de:de + 1]
        num = (jnp.dot(sx, wmx_ref[...], preferred_element_type=jnp.float32)
               + jnp.dot(sec, we_ref[...], preferred_element_type=jnp.float32))
        hn = num / jnp.maximum(cnt, 1.0)
        h = (jnp.dot(x_ref[...], wax_ref[...],
                     preferred_element_type=jnp.float32)
             + jnp.dot(hn, wah_ref[...], preferred_element_type=jnp.float32)
             + ba_ref[...])
        o_ref[...] = jnp.maximum(h, 0.0)

    return pl.pallas_call(
        body,
        grid=(n_pad // bn,),
        in_specs=[
            pl.BlockSpec((NC, bn, din), lambda i: (0, i, 0)),
            pl.BlockSpec((NC, bn, 16), lambda i: (0, i, 0)),
            pl.BlockSpec((bn, din), lambda i: (i, 0)),
            pl.BlockSpec((din, dout), lambda i: (0, 0)),
            pl.BlockSpec((16, dout), lambda i: (0, 0)),
            pl.BlockSpec((din, dout), lambda i: (0, 0)),
            pl.BlockSpec((dout, dout), lambda i: (0, 0)),
            pl.BlockSpec((1, dout), lambda i: (0, 0)),
        ],
        out_specs=pl.BlockSpec((bn, dout), lambda i: (i, 0)),
        out_shape=jax.ShapeDtypeStruct((n_pad, dout), jnp.float32),
    )(aggx, aggec, x2dp, wmx, wext, wax, wah, ba2)


def kernel(edge_index, nfeats, efeats, Wm, bm, Wa, ba):
    n = nfeats.shape[0]
    e = edge_index.shape[1]
    din = nfeats.shape[2]
    de = efeats.shape[2]
    dout = Wm.shape[0]

    x2d = nfeats.reshape(n, din)
    src = edge_index[0]
    dst = edge_index[1]

    e_pad = ((e + NW * K - 1) // (NW * K)) * (NW * K)
    npe = e_pad - e
    n_pad = ((n + 256 + 127) // 128) * 128
    trash = n_pad - n
    pad_ids = jnp.arange(npe, dtype=jnp.int32)
    srcp = jnp.concatenate([src, pad_ids % n])
    dstp = jnp.concatenate([dst, n + (pad_ids % trash)])
    ec = jnp.concatenate(
        [efeats.reshape(e, de),
         jnp.ones((e, 1), jnp.float32),
         jnp.zeros((e, 16 - de - 1), jnp.float32)], axis=1)
    ecp = jnp.concatenate([ec, jnp.zeros((npe, 16), jnp.float32)], axis=0)

    zr = n_pad // NS
    zx = jnp.zeros((zr, din), jnp.float32)
    zec = jnp.zeros((zr, 16), jnp.float32)

    aggx, aggec = _sc_aggregate(x2d, srcp, dstp, ecp, zx, zec, n_pad, e_pad)

    wmx = Wm[:, :din].T
    wext = jnp.concatenate(
        [Wm[:, din:].T, bm[None, :], jnp.zeros((16 - de - 1, dout))], axis=0)
    wax = Wa[:, :din].T
    wah = Wa[:, din:].T
    ba2 = ba[None, :]

    x2dp = jnp.concatenate([x2d, jnp.zeros((n_pad - n, din), jnp.float32)])
    out = _tc_apply(aggx, aggec, x2dp, wmx, wext, wax, wah, ba2, de, bn=1024)
    return out[:n].reshape(n, 1, dout)


# speedup vs baseline: 3.1330x; 3.1330x over previous
import functools

import jax
import jax.numpy as jnp
from jax import lax
from jax.experimental import pallas as pl
from jax.experimental.pallas import tpu as pltpu
from jax.experimental.pallas import tpu_sc as plsc

NC = 2
NS = 16
NW = NC * NS
K = 128


def _sc_aggregate(x2d, srcp, dstp, ec128, zx, n_pad, e_pad):
    din = x2d.shape[1]
    zr = n_pad // NS
    b_w = e_pad // NW
    n_iter = b_w // K
    nzc = zr // K

    mesh = plsc.VectorSubcoreMesh(core_axis_name="c", subcore_axis_name="s")

    @functools.partial(
        pl.kernel,
        out_type=(
            jax.ShapeDtypeStruct((NC * n_pad, din), jnp.float32),
            jax.ShapeDtypeStruct((NC * n_pad, din), jnp.float32),
        ),
        mesh=mesh,
        scratch_types=[
            pltpu.VMEM((K,), jnp.int32),
            pltpu.VMEM((K,), jnp.int32),
            pltpu.VMEM((K, din), jnp.float32),
            pltpu.VMEM_SHARED((n_pad, din), jnp.float32),
            pltpu.SemaphoreType.DMA,
        ],
    )
    def sc_agg(x_hbm, src_hbm, dst_hbm, ec_hbm, zx_hbm,
               outx_hbm, outec_hbm,
               srcv, dstv, rowsv, agg_sh, sem):
        cid = lax.axis_index("c")
        sid = lax.axis_index("s")
        wid = cid * NS + sid
        r0 = sid * zr
        o0 = cid * n_pad + r0
        base0 = wid * b_w

        pltpu.sync_copy(zx_hbm, rowsv)

        @pl.loop(0, nzc)
        def _(j):
            pltpu.sync_copy(rowsv, agg_sh.at[pl.ds(r0 + j * K, K)])

        plsc.subcore_barrier()

        @pl.loop(0, n_iter)
        def _(i):
            base = base0 + i * K
            pltpu.sync_copy(src_hbm.at[pl.ds(base, K)], srcv)
            pltpu.sync_copy(dst_hbm.at[pl.ds(base, K)], dstv)
            pltpu.async_copy(x_hbm.at[srcv], rowsv, sem).wait()
            pltpu.sync_copy(rowsv, agg_sh.at[dstv], add=True)

        plsc.subcore_barrier()

        @pl.loop(0, nzc)
        def _(j):
            pltpu.sync_copy(agg_sh.at[pl.ds(r0 + j * K, K)], rowsv)
            pltpu.sync_copy(rowsv, outx_hbm.at[pl.ds(o0 + j * K, K)])

        plsc.subcore_barrier()

        pltpu.sync_copy(zx_hbm, rowsv)

        @pl.loop(0, nzc)
        def _(j):
            pltpu.sync_copy(rowsv, agg_sh.at[pl.ds(r0 + j * K, K)])

        plsc.subcore_barrier()

        @pl.loop(0, n_iter)
        def _(i):
            base = base0 + i * K
            pltpu.sync_copy(dst_hbm.at[pl.ds(base, K)], dstv)
            pltpu.sync_copy(ec_hbm.at[pl.ds(base, K)], rowsv)
            pltpu.sync_copy(rowsv, agg_sh.at[dstv], add=True)

        plsc.subcore_barrier()

        @pl.loop(0, nzc)
        def _(j):
            pltpu.sync_copy(agg_sh.at[pl.ds(r0 + j * K, K)], rowsv)
            pltpu.sync_copy(rowsv, outec_hbm.at[pl.ds(o0 + j * K, K)])

    return sc_agg(x2d, srcp, dstp, ec128, zx)


def _tc_apply(aggx, aggec, x2dp, wmx, wext, wax, wah, ba2, de, bn):
    n_pad, din = x2dp.shape
    dout = wax.shape[1]

    def body(ax_ref, ae_ref, x_ref, wmx_ref, we_ref, wax_ref, wah_ref,
             ba_ref, o_ref):
        sx = ax_ref[0] + ax_ref[1]
        sec = ae_ref[0] + ae_ref[1]
        cnt = sec[:, de:de + 1]
        num = (jnp.dot(sx, wmx_ref[...], preferred_element_type=jnp.float32)
               + jnp.dot(sec, we_ref[...], preferred_element_type=jnp.float32))
        hn = num / jnp.maximum(cnt, 1.0)
        h = (jnp.dot(x_ref[...], wax_ref[...],
                     preferred_element_type=jnp.float32)
             + jnp.dot(hn, wah_ref[...], preferred_element_type=jnp.float32)
             + ba_ref[...])
        o_ref[...] = jnp.maximum(h, 0.0)

    return pl.pallas_call(
        body,
        grid=(n_pad // bn,),
        in_specs=[
            pl.BlockSpec((NC, bn, din), lambda i: (0, i, 0)),
            pl.BlockSpec((NC, bn, din), lambda i: (0, i, 0)),
            pl.BlockSpec((bn, din), lambda i: (i, 0)),
            pl.BlockSpec((din, dout), lambda i: (0, 0)),
            pl.BlockSpec((din, dout), lambda i: (0, 0)),
            pl.BlockSpec((din, dout), lambda i: (0, 0)),
            pl.BlockSpec((dout, dout), lambda i: (0, 0)),
            pl.BlockSpec((1, dout), lambda i: (0, 0)),
        ],
        out_specs=pl.BlockSpec((bn, dout), lambda i: (i, 0)),
        out_shape=jax.ShapeDtypeStruct((n_pad, dout), jnp.float32),
    )(aggx, aggec, x2dp, wmx, wext, wax, wah, ba2)


def kernel(edge_index, nfeats, efeats, Wm, bm, Wa, ba):
    n = nfeats.shape[0]
    e = edge_index.shape[1]
    din = nfeats.shape[2]
    de = efeats.shape[2]
    dout = Wm.shape[0]

    x2d = nfeats.reshape(n, din)
    src = edge_index[0]
    dst = edge_index[1]

    e_pad = ((e + NW * K - 1) // (NW * K)) * (NW * K)
    npe = e_pad - e
    n_pad = ((n + 256 + 127) // 128) * 128
    trash = n_pad - n
    pad_ids = jnp.arange(npe, dtype=jnp.int32)
    srcp = jnp.concatenate([src, pad_ids % n])
    dstp = jnp.concatenate([dst, n + (pad_ids % trash)])
    ec128 = jnp.concatenate(
        [efeats.reshape(e, de),
         jnp.ones((e, 1), jnp.float32),
         jnp.zeros((e, din - de - 1), jnp.float32)], axis=1)
    ec128 = jnp.concatenate(
        [ec128, jnp.zeros((npe, din), jnp.float32)], axis=0)

    zx = jnp.zeros((K, din), jnp.float32)

    aggx, aggec = _sc_aggregate(x2d, srcp, dstp, ec128, zx, n_pad, e_pad)
    aggx = aggx.reshape(NC, n_pad, din)
    aggec = aggec.reshape(NC, n_pad, din)

    wmx = Wm[:, :din].T
    wext = jnp.concatenate(
        [Wm[:, din:].T, bm[None, :], jnp.zeros((din - de - 1, dout))], axis=0)
    wax = Wa[:, :din].T
    wah = Wa[:, din:].T
    ba2 = ba[None, :]

    x2dp = jnp.concatenate([x2d, jnp.zeros((n_pad - n, din), jnp.float32)])
    out = _tc_apply(aggx, aggec, x2dp, wmx, wext, wax, wah, ba2, de, bn=1024)
    return out[:n].reshape(n, 1, dout)
